# baseline (speedup 1.0000x reference)
import functools

import jax
import jax.numpy as jnp
from jax import lax
from jax.experimental import pallas as pl
from jax.experimental.pallas import tpu as pltpu
from jax.experimental.pallas import tpu_sc as plsc

N_CB = 4
VOCAB = 100000
D = 64
BATCH = 16384

NUM_CORES = 2
NUM_SUBCORES = 16
NW = NUM_CORES * NUM_SUBCORES
B_PER_W = BATCH // NW
IDS_PER_W = B_PER_W * N_CB
CHUNK_IDS = 128
CHUNK_OUT = CHUNK_IDS // N_CB
N_CHUNKS_W = IDS_PER_W // CHUNK_IDS

_mesh = plsc.VectorSubcoreMesh(core_axis_name="c", subcore_axis_name="s")


@functools.partial(
    pl.kernel,
    mesh=_mesh,
    out_type=jax.ShapeDtypeStruct((BATCH // 2, 2 * D), jnp.float32),
    scratch_types=[
        pltpu.VMEM((N_CHUNKS_W, CHUNK_IDS), jnp.int32),
        pltpu.VMEM((N_CHUNKS_W, CHUNK_IDS), jnp.int32),
        pltpu.VMEM((N_CHUNKS_W, CHUNK_IDS), jnp.int32),
        pltpu.VMEM((CHUNK_IDS, 2 * D), jnp.float32),
        pltpu.VMEM((CHUNK_OUT // 2, 2 * D), jnp.float32),
        pltpu.SemaphoreType.DMA,
    ],
    compiler_params=pltpu.CompilerParams(needs_layout_passes=False),
)
def _sem_embed(ids_hbm, tab_hbm, out_hbm, idx_raw, idx_v, hcol_v, rows, outbuf, sem):
    wid = lax.axis_index("s") * NUM_CORES + lax.axis_index("c")

    pltpu.sync_copy(ids_hbm.at[pl.ds(wid * N_CHUNKS_W, N_CHUNKS_W)], idx_raw)

    lane = lax.iota(jnp.int32, 16)
    off = (lane % N_CB) * VOCAB
    for k in range(N_CHUNKS_W):
        for j in range(CHUNK_IDS // 16):
            sl = pl.ds(j * 16, 16)
            flat = idx_raw[k, sl] + off
            idx_v[k, sl] = lax.shift_right_logical(flat, 1)
            hcol_v[k, sl] = (flat & 1) * D

    def chunk_body(k, carry):
        pltpu.async_copy(tab_hbm.at[idx_v.at[k]], rows, sem).wait()
        kvec = k + jnp.zeros((16,), jnp.int32)
        for b in range(CHUNK_OUT):
            hb = [
                plsc.load_gather(hcol_v, [kvec, jnp.full((16,), 4 * b + i, jnp.int32)])
                for i in range(N_CB)
            ]
            half = (b % 2) * D
            for d in range(D // 16):
                col = 16 * d + lane
                acc = plsc.load_gather(
                    rows, [jnp.full((16,), 4 * b, jnp.int32), hb[0] + col]
                )
                for i in range(1, N_CB):
                    acc = acc + plsc.load_gather(
                        rows, [jnp.full((16,), 4 * b + i, jnp.int32), hb[i] + col]
                    )
                outbuf[b // 2, pl.ds(half + 16 * d, 16)] = acc
        pltpu.sync_copy(
            outbuf,
            out_hbm.at[pl.ds(wid * (B_PER_W // 2) + k * (CHUNK_OUT // 2), CHUNK_OUT // 2)],
        )
        return carry

    lax.fori_loop(0, N_CHUNKS_W, chunk_body, 0)


def kernel(semantic_ids, tables):
    ids_il = semantic_ids.astype(jnp.int32).reshape(NW * N_CHUNKS_W, CHUNK_IDS)
    tab2 = tables.reshape(N_CB * VOCAB // 2, 2 * D)
    out2 = _sem_embed(ids_il, tab2)
    return out2.reshape(BATCH, D)


# speedup vs baseline: 2.0941x; 2.0941x over previous
import functools

import jax
import jax.numpy as jnp
from jax import lax
from jax.experimental import pallas as pl
from jax.experimental.pallas import tpu as pltpu
from jax.experimental.pallas import tpu_sc as plsc

N_CB = 4
VOCAB = 100000
D = 64
BATCH = 16384

NUM_CORES = 2
NUM_SUBCORES = 16
NW = NUM_CORES * NUM_SUBCORES
E_PER_W = D // NW
CHUNK = 4096
N_CHUNKS = BATCH // CHUNK
GRP = 16
UNROLL = 16

_mesh = plsc.VectorSubcoreMesh(core_axis_name="c", subcore_axis_name="s")


@functools.partial(
    pl.kernel,
    mesh=_mesh,
    out_type=jax.ShapeDtypeStruct((D, BATCH), jnp.float32),
    scratch_types=[
        pltpu.VMEM((1, VOCAB), jnp.float32),
        pltpu.VMEM((1, BATCH), jnp.float32),
        pltpu.VMEM((2, CHUNK), jnp.int32),
        pltpu.SemaphoreType.DMA,
        pltpu.SemaphoreType.DMA,
        pltpu.SemaphoreType.DMA,
    ],
    compiler_params=pltpu.CompilerParams(needs_layout_passes=False),
)
def _sem_embed(ids_hbm, tab_hbm, out_hbm, vocab, acc, idc, sem_v, sem_i0, sem_i1):
    wid = lax.axis_index("s") * NUM_CORES + lax.axis_index("c")
    id_sems = (sem_i0, sem_i1)

    for e_local in range(E_PER_W):
        e = E_PER_W * wid + e_local
        for i in range(N_CB):
            vcopy = pltpu.async_copy(tab_hbm.at[pl.ds(i * D + e, 1)], vocab, sem_v)
            pltpu.async_copy(
                ids_hbm.at[pl.ds(i, 1), pl.ds(0, CHUNK)], idc.at[pl.ds(0, 1)], id_sems[0]
            )
            vcopy.wait()

            def gathers(c, buf, first):
                base = c * CHUNK

                def grp_body(g, carry):
                    start = g * (GRP * UNROLL)
                    zero = jnp.zeros((GRP,), jnp.int32)
                    for u in range(UNROLL):
                        o = start + u * GRP
                        idx = idc[buf, pl.ds(o, GRP)]
                        val = plsc.load_gather(vocab, [zero, idx])
                        if first:
                            acc[0, pl.ds(base + o, GRP)] = val
                        else:
                            acc[0, pl.ds(base + o, GRP)] = acc[0, pl.ds(base + o, GRP)] + val
                    return carry

                lax.fori_loop(0, CHUNK // (GRP * UNROLL), grp_body, 0)

            for c in range(N_CHUNKS):
                if c + 1 < N_CHUNKS:
                    pltpu.async_copy(
                        ids_hbm.at[pl.ds(i, 1), pl.ds((c + 1) * CHUNK, CHUNK)],
                        idc.at[pl.ds((c + 1) % 2, 1)],
                        id_sems[(c + 1) % 2],
                    )
                pltpu.make_async_copy(
                    ids_hbm.at[pl.ds(i, 1), pl.ds(c * CHUNK, CHUNK)],
                    idc.at[pl.ds(c % 2, 1)],
                    id_sems[c % 2],
                ).wait()
                gathers(c, c % 2, first=(i == 0))

        pltpu.sync_copy(acc, out_hbm.at[pl.ds(e, 1)])


def kernel(semantic_ids, tables):
    tab_t = jnp.transpose(tables, (0, 2, 1)).reshape(N_CB * D, VOCAB)
    ids_t = jnp.transpose(semantic_ids).astype(jnp.int32)
    out_t = _sem_embed(ids_t, tab_t)
    return jnp.transpose(out_t)
